# baseline (speedup 1.0000x reference)
import functools

import jax
import jax.numpy as jnp
import numpy as np
from jax import lax
from jax.experimental import pallas as pl
from jax.experimental.pallas import tpu as pltpu
from jax.experimental.pallas import tpu_sc as plsc

_N_BINS = 15


def _conf_body(logits_ref, labels_ref, out_ref):
    x = logits_ref[...]
    m = jnp.max(x, axis=1, keepdims=True)
    s = jnp.sum(jnp.exp(x - m), axis=1, keepdims=True)
    conf = 1.0 / s
    col = lax.broadcasted_iota(jnp.int32, x.shape, 1)
    cand = jnp.where(x == m, col, x.shape[1])
    pred = jnp.min(cand, axis=1, keepdims=True)
    acc = pred == labels_ref[0]
    out_ref[0] = jnp.where(acc, conf, -conf)


def _tc_stage(logits, labels, block_rows):
    n, c = logits.shape
    nblk = n // block_rows
    out = pl.pallas_call(
        _conf_body,
        grid=(nblk,),
        in_specs=[
            pl.BlockSpec((block_rows, c), lambda i: (i, 0)),
            pl.BlockSpec((1, block_rows, 1), lambda i: (i, 0, 0)),
        ],
        out_specs=pl.BlockSpec((1, block_rows, 1), lambda i: (i, 0, 0)),
        out_shape=jax.ShapeDtypeStruct((nblk, block_rows, 1), jnp.float32),
    )(logits, labels.reshape(nblk, block_rows, 1))
    return out.reshape(n)


_NW = 16
_L = 16


def _sc_stage(sconf):
    n = sconf.shape[0]
    ch_full = ((n + _NW - 1) // _NW + _L - 1) // _L * _L
    ch_last = n - ch_full * (_NW - 1)
    assert ch_last > 0 and ch_last % _L == 0 and (ch_full * (_NW - 1)) % 8 == 0
    nvec_full = ch_full // _L
    nvec_last = ch_last // _L

    uppers = [float(np.float32(i) / np.float32(_N_BINS))
              for i in range(1, _N_BINS)] + [1.0]
    inv_n = 1.0 / n

    mesh = plsc.VectorSubcoreMesh(core_axis_name="c", subcore_axis_name="s",
                                  num_cores=1)

    @functools.partial(
        pl.kernel,
        mesh=mesh,
        out_type=jax.ShapeDtypeStruct((_L,), jnp.float32),
        compiler_params=pltpu.CompilerParams(needs_layout_passes=False),
        scratch_types=[
            pltpu.VMEM((ch_full,), jnp.float32),
            pltpu.VMEM((3 * 16 * _L,), jnp.float32),
            pltpu.VMEM((_NW, 3 * 16 * _L), jnp.float32),
            pltpu.VMEM((_L,), jnp.float32),
            pltpu.VMEM_SHARED((_NW, 3 * 16 * _L), jnp.float32),
        ],
    )
    def sc_kernel(sconf_hbm, out_hbm, chunk_v, part_v, comb_v, outv_v, shared):
        wid = lax.axis_index("s")
        lane = lax.iota(jnp.int32, _L)
        zeros = jnp.zeros((_L,), jnp.float32)
        ones = jnp.ones((_L,), jnp.float32)
        idx_z = jnp.zeros((_L,), jnp.int32)

        def zk(k, _):
            part_v[pl.ds(k * _L, _L)] = zeros
            return 0
        lax.fori_loop(0, 3 * 16, zk, 0)

        base = wid * ch_full

        @pl.when(wid < _NW - 1)
        def _():
            pltpu.sync_copy(sconf_hbm.at[pl.ds(base, ch_full)], chunk_v)

        @pl.when(wid == _NW - 1)
        def _():
            pltpu.sync_copy(sconf_hbm.at[pl.ds(base, ch_last)],
                            chunk_v.at[pl.ds(0, ch_last)])

        nvec = jnp.where(wid == _NW - 1, nvec_last, nvec_full)

        def body(t, _):
            v = chunk_v[pl.ds(t * _L, _L)]
            conf = jnp.abs(v)
            accf = jnp.where(v > 0, 1.0, 0.0)
            idx = jnp.zeros((_L,), jnp.int32)
            one_i = jnp.ones((_L,), jnp.int32)
            for u in uppers:
                idx = idx + jnp.where(conf > u, one_i, idx_z)
            off = idx * _L + lane
            plsc.addupdate_scatter(part_v, [off], ones)
            plsc.addupdate_scatter(part_v, [off + 256], conf)
            plsc.addupdate_scatter(part_v, [off + 512], accf)
            return 0
        lax.fori_loop(0, nvec, body, 0)

        pltpu.sync_copy(part_v, shared.at[wid])
        plsc.subcore_barrier()

        @pl.when(wid == 0)
        def _():
            pltpu.sync_copy(shared, comb_v)

            def comb_k(k, _):
                def add_w(w, t):
                    return t + comb_v[w, pl.ds(k * _L, _L)]
                tot = lax.fori_loop(1, _NW, add_w, comb_v[0, pl.ds(k * _L, _L)])
                part_v[pl.ds(k * _L, _L)] = tot
                return 0
            lax.fori_loop(0, 3 * 16, comb_k, 0)

            vecs = []
            for a in range(3):
                vec = zeros
                for b in range(16):
                    s = jnp.sum(part_v[pl.ds(a * 256 + b * _L, _L)])
                    vec = jnp.where(lane == b, s, vec)
                vecs.append(vec)
            cntv, confv, accv = vecs
            denom = jnp.maximum(cntv, 1.0)
            avg_c = confv / denom
            avg_a = accv / denom
            prop = cntv * inv_n
            contrib = jnp.where(cntv > 0.0,
                                jnp.abs(avg_c - avg_a) * prop, 0.0)
            contrib = jnp.where(lane < _N_BINS, contrib, 0.0)
            outv_v[...] = jnp.sum(contrib) + zeros
            pltpu.sync_copy(outv_v, out_hbm)

    return sc_kernel(sconf)


def kernel(logits, labels):
    sconf = _tc_stage(logits, labels, block_rows=2000)
    out = _sc_stage(sconf)
    return out[:1]


# speedup vs baseline: 2.2814x; 2.2814x over previous
import functools

import jax
import jax.numpy as jnp
import numpy as np
from jax import lax
from jax.experimental import pallas as pl
from jax.experimental.pallas import tpu as pltpu
from jax.experimental.pallas import tpu_sc as plsc

_N_BINS = 15
_BR = 2048


def _conf_body(n_rows, logits_ref, labels_ref, out_ref):
    x = logits_ref[...]
    br, c = x.shape
    m = jnp.max(x, axis=1, keepdims=True)
    e = jnp.exp(x - m)
    ind = jnp.where(e == 1.0, 1.0, 0.0)
    ones_row = jnp.ones((1, c), jnp.float32)
    iota_row = lax.broadcasted_iota(jnp.int32, (1, c), 1).astype(jnp.float32)
    dn = (((1,), (1,)), ((), ()))
    s = lax.dot_general(ones_row, e, dimension_numbers=dn,
                        preferred_element_type=jnp.float32)
    am = lax.dot_general(iota_row, ind, dimension_numbers=dn,
                         preferred_element_type=jnp.float32)
    conf = 1.0 / s
    labf = labels_ref[...].astype(jnp.float32).reshape(1, br)
    acc = am == labf
    signed = jnp.where(acc, conf, -conf)
    glob = pl.program_id(0) * br + lax.broadcasted_iota(jnp.int32, (1, br), 1)
    out_ref[...] = jnp.where(glob < n_rows, signed, 2.0).reshape(br)


def _tc_stage(logits, labels):
    n, c = logits.shape
    grid = (n + _BR - 1) // _BR
    n_pad = grid * _BR
    return pl.pallas_call(
        functools.partial(_conf_body, n),
        grid=(grid,),
        in_specs=[
            pl.BlockSpec((_BR, c), lambda i: (i, 0)),
            pl.BlockSpec((_BR,), lambda i: (i,)),
        ],
        out_specs=pl.BlockSpec((_BR,), lambda i: (i,)),
        out_shape=jax.ShapeDtypeStruct((n_pad,), jnp.float32),
    )(logits, labels)


_NW = 16
_L = 16


def _sc_stage(sconf, n_real):
    n = sconf.shape[0]
    ch = n // _NW
    nvec = ch // _L
    assert ch * _NW == n and nvec * _L == ch and ch % 8 == 0

    uppers = [float(np.float32(i) / np.float32(_N_BINS))
              for i in range(1, _N_BINS)] + [1.0]
    inv_n = 1.0 / n_real

    mesh = plsc.VectorSubcoreMesh(core_axis_name="c", subcore_axis_name="s",
                                  num_cores=1)

    @functools.partial(
        pl.kernel,
        mesh=mesh,
        out_type=jax.ShapeDtypeStruct((_L,), jnp.float32),
        compiler_params=pltpu.CompilerParams(needs_layout_passes=False),
        scratch_types=[
            pltpu.VMEM((ch,), jnp.float32),
            pltpu.VMEM((3 * 16 * _L,), jnp.float32),
            pltpu.VMEM((_NW, 3 * 16 * _L), jnp.float32),
            pltpu.VMEM((_L,), jnp.float32),
            pltpu.VMEM_SHARED((_NW, 3 * 16 * _L), jnp.float32),
        ],
    )
    def sc_kernel(sconf_hbm, out_hbm, chunk_v, part_v, comb_v, outv_v, shared):
        wid = lax.axis_index("s")
        lane = lax.iota(jnp.int32, _L)
        zeros = jnp.zeros((_L,), jnp.float32)
        ones = jnp.ones((_L,), jnp.float32)
        one_i = jnp.ones((_L,), jnp.int32)
        zero_i = jnp.zeros((_L,), jnp.int32)

        def zk(k, _):
            part_v[pl.ds(k * _L, _L)] = zeros
            return 0
        lax.fori_loop(0, 3 * 16, zk, 0)

        pltpu.sync_copy(sconf_hbm.at[pl.ds(wid * ch, ch)], chunk_v)

        def body(t, _):
            v = chunk_v[pl.ds(t * _L, _L)]
            conf = jnp.abs(v)
            accf = jnp.where(v > 0, 1.0, 0.0)
            idx = zero_i
            for u in uppers:
                idx = idx + jnp.where(conf > u, one_i, zero_i)
            off = idx * _L + lane
            plsc.addupdate_scatter(part_v, [off], ones)
            plsc.addupdate_scatter(part_v, [off + 256], conf)
            plsc.addupdate_scatter(part_v, [off + 512], accf)
            return 0
        lax.fori_loop(0, nvec, body, 0)

        pltpu.sync_copy(part_v, shared.at[wid])
        plsc.subcore_barrier()

        @pl.when(wid == 0)
        def _():
            pltpu.sync_copy(shared, comb_v)

            def comb_k(k, _):
                def add_w(w, t):
                    return t + comb_v[w, pl.ds(k * _L, _L)]
                tot = lax.fori_loop(1, _NW, add_w, comb_v[0, pl.ds(k * _L, _L)])
                part_v[pl.ds(k * _L, _L)] = tot
                return 0
            lax.fori_loop(0, 3 * 16, comb_k, 0)

            vecs = []
            for a in range(3):
                vec = zeros
                for b in range(16):
                    s = jnp.sum(part_v[pl.ds(a * 256 + b * _L, _L)])
                    vec = jnp.where(lane == b, s, vec)
                vecs.append(vec)
            cntv, confv, accv = vecs
            denom = jnp.maximum(cntv, 1.0)
            avg_c = confv / denom
            avg_a = accv / denom
            prop = cntv * inv_n
            contrib = jnp.where(cntv > 0.0,
                                jnp.abs(avg_c - avg_a) * prop, 0.0)
            contrib = jnp.where(lane < _N_BINS, contrib, 0.0)
            outv_v[...] = jnp.sum(contrib) + zeros
            pltpu.sync_copy(outv_v, out_hbm)

    return sc_kernel(sconf)


def kernel(logits, labels):
    sconf = _tc_stage(logits, labels)
    out = _sc_stage(sconf, logits.shape[0])
    return out[:1]


# speedup vs baseline: 4.5882x; 2.0112x over previous
import functools

import jax
import jax.numpy as jnp
import numpy as np
from jax import lax
from jax.experimental import pallas as pl
from jax.experimental.pallas import tpu as pltpu
from jax.experimental.pallas import tpu_sc as plsc

_N_BINS = 15
_BR = 2048


def _conf_body(n_rows, logits_ref, labels_ref, out_ref):
    x = logits_ref[...]
    c, b = x.shape
    m = jnp.max(x, axis=0, keepdims=True)
    e = jnp.exp(x - m)
    ind = jnp.where(e == 1.0, 1.0, 0.0)
    ones_row = jnp.ones((1, c), jnp.float32)
    iota_row = lax.broadcasted_iota(jnp.int32, (1, c), 1).astype(jnp.float32)
    dn = (((1,), (0,)), ((), ()))
    s = lax.dot_general(ones_row, e, dimension_numbers=dn,
                        preferred_element_type=jnp.float32)
    am = lax.dot_general(iota_row, ind, dimension_numbers=dn,
                         preferred_element_type=jnp.float32)
    conf = 1.0 / s
    labf = labels_ref[...].astype(jnp.float32).reshape(1, b)
    acc = am == labf
    signed = jnp.where(acc, conf, -conf)
    glob = pl.program_id(0) * b + lax.broadcasted_iota(jnp.int32, (1, b), 1)
    out_ref[...] = jnp.where(glob < n_rows, signed, 2.0).reshape(b)


def _tc_stage(logits, labels):
    n, c = logits.shape
    grid = (n + _BR - 1) // _BR
    n_pad = grid * _BR
    return pl.pallas_call(
        functools.partial(_conf_body, n),
        grid=(grid,),
        in_specs=[
            pl.BlockSpec((c, _BR), lambda i: (0, i)),
            pl.BlockSpec((_BR,), lambda i: (i,)),
        ],
        out_specs=pl.BlockSpec((_BR,), lambda i: (i,)),
        out_shape=jax.ShapeDtypeStruct((n_pad,), jnp.float32),
    )(logits.T, labels)


_NW = 16
_L = 16


def _sc_stage(sconf, n_real):
    n = sconf.shape[0]
    ch = n // _NW
    nvec = ch // _L
    assert ch * _NW == n and nvec * _L == ch and ch % 8 == 0

    uppers = [float(np.float32(i) / np.float32(_N_BINS))
              for i in range(1, _N_BINS)] + [1.0]
    inv_n = 1.0 / n_real

    mesh = plsc.VectorSubcoreMesh(core_axis_name="c", subcore_axis_name="s",
                                  num_cores=1)

    @functools.partial(
        pl.kernel,
        mesh=mesh,
        out_type=jax.ShapeDtypeStruct((_L,), jnp.float32),
        compiler_params=pltpu.CompilerParams(needs_layout_passes=False),
        scratch_types=[
            pltpu.VMEM((ch,), jnp.float32),
            pltpu.VMEM((3 * 16 * _L,), jnp.float32),
            pltpu.VMEM((_NW, 3 * 16 * _L), jnp.float32),
            pltpu.VMEM((_L,), jnp.float32),
            pltpu.VMEM_SHARED((_NW, 3 * 16 * _L), jnp.float32),
        ],
    )
    def sc_kernel(sconf_hbm, out_hbm, chunk_v, part_v, comb_v, outv_v, shared):
        wid = lax.axis_index("s")
        lane = lax.iota(jnp.int32, _L)
        zeros = jnp.zeros((_L,), jnp.float32)
        ones = jnp.ones((_L,), jnp.float32)
        one_i = jnp.ones((_L,), jnp.int32)
        zero_i = jnp.zeros((_L,), jnp.int32)

        def zk(k, _):
            part_v[pl.ds(k * _L, _L)] = zeros
            return 0
        lax.fori_loop(0, 3 * 16, zk, 0)

        pltpu.sync_copy(sconf_hbm.at[pl.ds(wid * ch, ch)], chunk_v)

        def body(t, _):
            v = chunk_v[pl.ds(t * _L, _L)]
            conf = jnp.abs(v)
            accf = jnp.where(v > 0, 1.0, 0.0)
            idx = zero_i
            for u in uppers:
                idx = idx + jnp.where(conf > u, one_i, zero_i)
            off = idx * _L + lane
            plsc.addupdate_scatter(part_v, [off], ones)
            plsc.addupdate_scatter(part_v, [off + 256], conf)
            plsc.addupdate_scatter(part_v, [off + 512], accf)
            return 0
        lax.fori_loop(0, nvec, body, 0)

        pltpu.sync_copy(part_v, shared.at[wid])
        plsc.subcore_barrier()

        @pl.when(wid == 0)
        def _():
            pltpu.sync_copy(shared, comb_v)

            def comb_k(k, _):
                def add_w(w, t):
                    return t + comb_v[w, pl.ds(k * _L, _L)]
                tot = lax.fori_loop(1, _NW, add_w, comb_v[0, pl.ds(k * _L, _L)])
                part_v[pl.ds(k * _L, _L)] = tot
                return 0
            lax.fori_loop(0, 3 * 16, comb_k, 0)

            vecs = []
            for a in range(3):
                vec = zeros
                for b in range(16):
                    s = jnp.sum(part_v[pl.ds(a * 256 + b * _L, _L)])
                    vec = jnp.where(lane == b, s, vec)
                vecs.append(vec)
            cntv, confv, accv = vecs
            denom = jnp.maximum(cntv, 1.0)
            avg_c = confv / denom
            avg_a = accv / denom
            prop = cntv * inv_n
            contrib = jnp.where(cntv > 0.0,
                                jnp.abs(avg_c - avg_a) * prop, 0.0)
            contrib = jnp.where(lane < _N_BINS, contrib, 0.0)
            outv_v[...] = jnp.sum(contrib) + zeros
            pltpu.sync_copy(outv_v, out_hbm)

    return sc_kernel(sconf)


def kernel(logits, labels):
    sconf = _tc_stage(logits, labels)
    out = _sc_stage(sconf, logits.shape[0])
    return out[:1]


# speedup vs baseline: 7.8565x; 1.7123x over previous
import functools

import jax
import jax.numpy as jnp
import numpy as np
from jax import lax
from jax.experimental import pallas as pl
from jax.experimental.pallas import tpu as pltpu
from jax.experimental.pallas import tpu_sc as plsc

_N_BINS = 15
_BR = 8192


def _conf_body(n_rows, logits_ref, labels_ref, out_ref):
    x = logits_ref[...]
    c, b = x.shape
    m = jnp.max(x, axis=0, keepdims=True)
    e = jnp.exp(x - m)
    ind = jnp.where(e == 1.0, 1.0, 0.0)
    ones_row = jnp.ones((1, c), jnp.float32)
    iota_row = lax.broadcasted_iota(jnp.int32, (1, c), 1).astype(jnp.float32)
    dn = (((1,), (0,)), ((), ()))
    s = lax.dot_general(ones_row, e, dimension_numbers=dn,
                        preferred_element_type=jnp.float32)
    am = lax.dot_general(iota_row, ind, dimension_numbers=dn,
                         preferred_element_type=jnp.float32)
    conf = 1.0 / s
    labf = labels_ref[...].astype(jnp.float32).reshape(1, b)
    acc = am == labf
    signed = jnp.where(acc, conf, -conf)
    glob = pl.program_id(0) * b + lax.broadcasted_iota(jnp.int32, (1, b), 1)
    out_ref[...] = jnp.where(glob < n_rows, signed, 2.0).reshape(b)


def _tc_stage(logits, labels):
    n, c = logits.shape
    grid = (n + _BR - 1) // _BR
    n_pad = grid * _BR
    return pl.pallas_call(
        functools.partial(_conf_body, n),
        grid=(grid,),
        in_specs=[
            pl.BlockSpec((c, _BR), lambda i: (0, i)),
            pl.BlockSpec((_BR,), lambda i: (i,)),
        ],
        out_specs=pl.BlockSpec((_BR,), lambda i: (i,)),
        out_shape=jax.ShapeDtypeStruct((n_pad,), jnp.float32),
    )(logits.T, labels)


_NW = 16
_L = 16


def _sc_stage(sconf, n_real):
    n = sconf.shape[0]
    ch = n // _NW
    nvec = ch // _L
    assert ch * _NW == n and nvec * _L == ch and ch % 8 == 0

    uppers = [float(np.float32(i) / np.float32(_N_BINS))
              for i in range(1, _N_BINS)] + [1.0]
    inv_n = 1.0 / n_real

    mesh = plsc.VectorSubcoreMesh(core_axis_name="c", subcore_axis_name="s",
                                  num_cores=1)

    @functools.partial(
        pl.kernel,
        mesh=mesh,
        out_type=jax.ShapeDtypeStruct((_L,), jnp.float32),
        compiler_params=pltpu.CompilerParams(needs_layout_passes=False),
        scratch_types=[
            pltpu.VMEM((ch,), jnp.float32),
            pltpu.VMEM((3 * 16 * _L,), jnp.float32),
            pltpu.VMEM((_NW, 3 * 16 * _L), jnp.float32),
            pltpu.VMEM((_L,), jnp.float32),
            pltpu.VMEM_SHARED((_NW, 3 * 16 * _L), jnp.float32),
        ],
    )
    def sc_kernel(sconf_hbm, out_hbm, chunk_v, part_v, comb_v, outv_v, shared):
        wid = lax.axis_index("s")
        lane = lax.iota(jnp.int32, _L)
        zeros = jnp.zeros((_L,), jnp.float32)
        ones = jnp.ones((_L,), jnp.float32)
        one_i = jnp.ones((_L,), jnp.int32)
        zero_i = jnp.zeros((_L,), jnp.int32)

        def zk(k, _):
            part_v[pl.ds(k * _L, _L)] = zeros
            return 0
        lax.fori_loop(0, 3 * 16, zk, 0)

        pltpu.sync_copy(sconf_hbm.at[pl.ds(wid * ch, ch)], chunk_v)

        def body(t, _):
            v = chunk_v[pl.ds(t * _L, _L)]
            conf = jnp.abs(v)
            accf = jnp.where(v > 0, 1.0, 0.0)
            idx = zero_i
            for u in uppers:
                idx = idx + jnp.where(conf > u, one_i, zero_i)
            off = idx * _L + lane
            plsc.addupdate_scatter(part_v, [off], ones)
            plsc.addupdate_scatter(part_v, [off + 256], conf)
            plsc.addupdate_scatter(part_v, [off + 512], accf)
            return 0
        lax.fori_loop(0, nvec, body, 0)

        pltpu.sync_copy(part_v, shared.at[wid])
        plsc.subcore_barrier()

        @pl.when(wid == 0)
        def _():
            pltpu.sync_copy(shared, comb_v)

            def comb_k(k, _):
                def add_w(w, t):
                    return t + comb_v[w, pl.ds(k * _L, _L)]
                tot = lax.fori_loop(1, _NW, add_w, comb_v[0, pl.ds(k * _L, _L)])
                part_v[pl.ds(k * _L, _L)] = tot
                return 0
            lax.fori_loop(0, 3 * 16, comb_k, 0)

            vecs = []
            for a in range(3):
                vec = zeros
                for b in range(16):
                    s = jnp.sum(part_v[pl.ds(a * 256 + b * _L, _L)])
                    vec = jnp.where(lane == b, s, vec)
                vecs.append(vec)
            cntv, confv, accv = vecs
            denom = jnp.maximum(cntv, 1.0)
            avg_c = confv / denom
            avg_a = accv / denom
            prop = cntv * inv_n
            contrib = jnp.where(cntv > 0.0,
                                jnp.abs(avg_c - avg_a) * prop, 0.0)
            contrib = jnp.where(lane < _N_BINS, contrib, 0.0)
            outv_v[...] = jnp.sum(contrib) + zeros
            pltpu.sync_copy(outv_v, out_hbm)

    return sc_kernel(sconf)


def kernel(logits, labels):
    sconf = _tc_stage(logits, labels)
    out = _sc_stage(sconf, logits.shape[0])
    return out[:1]


# speedup vs baseline: 8.9870x; 1.1439x over previous
import functools

import jax
import jax.numpy as jnp
import numpy as np
from jax import lax
from jax.experimental import pallas as pl
from jax.experimental.pallas import tpu as pltpu
from jax.experimental.pallas import tpu_sc as plsc

_N_BINS = 15
_BR = 16384


def _conf_body(n_rows, logits_ref, labels_ref, out_ref):
    x = logits_ref[...]
    c, b = x.shape
    m = jnp.max(x, axis=0, keepdims=True)
    e = jnp.exp(x - m)
    ind = jnp.where(e == 1.0, 1.0, 0.0)
    ones_row = jnp.ones((1, c), jnp.float32)
    iota_row = lax.broadcasted_iota(jnp.int32, (1, c), 1).astype(jnp.float32)
    dn = (((1,), (0,)), ((), ()))
    s = lax.dot_general(ones_row, e, dimension_numbers=dn,
                        preferred_element_type=jnp.float32)
    am = lax.dot_general(iota_row, ind, dimension_numbers=dn,
                         preferred_element_type=jnp.float32)
    conf = 1.0 / s
    labf = labels_ref[...].astype(jnp.float32).reshape(1, b)
    acc = am == labf
    signed = jnp.where(acc, conf, -conf)
    glob = pl.program_id(0) * b + lax.broadcasted_iota(jnp.int32, (1, b), 1)
    out_ref[...] = jnp.where(glob < n_rows, signed, 2.0).reshape(b)


def _tc_stage(logits, labels):
    n, c = logits.shape
    grid = (n + _BR - 1) // _BR
    n_pad = grid * _BR
    return pl.pallas_call(
        functools.partial(_conf_body, n),
        grid=(grid,),
        in_specs=[
            pl.BlockSpec((c, _BR), lambda i: (0, i)),
            pl.BlockSpec((_BR,), lambda i: (i,)),
        ],
        out_specs=pl.BlockSpec((_BR,), lambda i: (i,)),
        out_shape=jax.ShapeDtypeStruct((n_pad,), jnp.float32),
    )(logits.T, labels)


_NW = 16
_L = 16


def _sc_stage(sconf, n_real):
    n = sconf.shape[0]
    ch = n // _NW
    nvec = ch // _L
    assert ch * _NW == n and nvec * _L == ch and ch % 8 == 0

    uppers = [float(np.float32(i) / np.float32(_N_BINS))
              for i in range(1, _N_BINS)] + [1.0]
    inv_n = 1.0 / n_real

    mesh = plsc.VectorSubcoreMesh(core_axis_name="c", subcore_axis_name="s",
                                  num_cores=1)

    @functools.partial(
        pl.kernel,
        mesh=mesh,
        out_type=jax.ShapeDtypeStruct((_L,), jnp.float32),
        compiler_params=pltpu.CompilerParams(needs_layout_passes=False),
        scratch_types=[
            pltpu.VMEM((ch,), jnp.float32),
            pltpu.VMEM((3 * 16 * _L,), jnp.float32),
            pltpu.VMEM((_NW, 3 * 16 * _L), jnp.float32),
            pltpu.VMEM((_L,), jnp.float32),
            pltpu.VMEM_SHARED((_NW, 3 * 16 * _L), jnp.float32),
        ],
    )
    def sc_kernel(sconf_hbm, out_hbm, chunk_v, part_v, comb_v, outv_v, shared):
        wid = lax.axis_index("s")
        lane = lax.iota(jnp.int32, _L)
        zeros = jnp.zeros((_L,), jnp.float32)
        ones = jnp.ones((_L,), jnp.float32)
        one_i = jnp.ones((_L,), jnp.int32)
        zero_i = jnp.zeros((_L,), jnp.int32)

        def zk(k, _):
            part_v[pl.ds(k * _L, _L)] = zeros
            return 0
        lax.fori_loop(0, 3 * 16, zk, 0)

        pltpu.sync_copy(sconf_hbm.at[pl.ds(wid * ch, ch)], chunk_v)

        def body(t, _):
            v = chunk_v[pl.ds(t * _L, _L)]
            conf = jnp.abs(v)
            accf = jnp.where(v > 0, 1.0, 0.0)
            idx = zero_i
            for u in uppers:
                idx = idx + jnp.where(conf > u, one_i, zero_i)
            off = idx * _L + lane
            plsc.addupdate_scatter(part_v, [off], ones)
            plsc.addupdate_scatter(part_v, [off + 256], conf)
            plsc.addupdate_scatter(part_v, [off + 512], accf)
            return 0
        lax.fori_loop(0, nvec, body, 0)

        pltpu.sync_copy(part_v, shared.at[wid])
        plsc.subcore_barrier()

        @pl.when(wid == 0)
        def _():
            pltpu.sync_copy(shared, comb_v)

            def comb_k(k, _):
                def add_w(w, t):
                    return t + comb_v[w, pl.ds(k * _L, _L)]
                tot = lax.fori_loop(1, _NW, add_w, comb_v[0, pl.ds(k * _L, _L)])
                part_v[pl.ds(k * _L, _L)] = tot
                return 0
            lax.fori_loop(0, 3 * 16, comb_k, 0)

            vecs = []
            for a in range(3):
                vec = zeros
                for b in range(16):
                    s = jnp.sum(part_v[pl.ds(a * 256 + b * _L, _L)])
                    vec = jnp.where(lane == b, s, vec)
                vecs.append(vec)
            cntv, confv, accv = vecs
            denom = jnp.maximum(cntv, 1.0)
            avg_c = confv / denom
            avg_a = accv / denom
            prop = cntv * inv_n
            contrib = jnp.where(cntv > 0.0,
                                jnp.abs(avg_c - avg_a) * prop, 0.0)
            contrib = jnp.where(lane < _N_BINS, contrib, 0.0)
            outv_v[...] = jnp.sum(contrib) + zeros
            pltpu.sync_copy(outv_v, out_hbm)

    return sc_kernel(sconf)


def kernel(logits, labels):
    sconf = _tc_stage(logits, labels)
    out = _sc_stage(sconf, logits.shape[0])
    return out[:1]


# speedup vs baseline: 9.6468x; 1.0734x over previous
import functools

import jax
import jax.numpy as jnp
import numpy as np
from jax import lax
from jax.experimental import pallas as pl
from jax.experimental.pallas import tpu as pltpu
from jax.experimental.pallas import tpu_sc as plsc

_N_BINS = 15
_BR = 32768


def _conf_body(n_rows, logits_ref, labels_ref, out_ref):
    x = logits_ref[...]
    c, b = x.shape
    m = jnp.max(x, axis=0, keepdims=True)
    e = jnp.exp(x - m)
    ind = jnp.where(e == 1.0, 1.0, 0.0)
    ones_row = jnp.ones((1, c), jnp.float32)
    iota_row = lax.broadcasted_iota(jnp.int32, (1, c), 1).astype(jnp.float32)
    dn = (((1,), (0,)), ((), ()))
    s = lax.dot_general(ones_row, e, dimension_numbers=dn,
                        preferred_element_type=jnp.float32)
    am = lax.dot_general(iota_row, ind, dimension_numbers=dn,
                         preferred_element_type=jnp.float32)
    conf = 1.0 / s
    labf = labels_ref[...].astype(jnp.float32).reshape(1, b)
    acc = am == labf
    signed = jnp.where(acc, conf, -conf)
    glob = pl.program_id(0) * b + lax.broadcasted_iota(jnp.int32, (1, b), 1)
    out_ref[...] = jnp.where(glob < n_rows, signed, 2.0).reshape(b)


def _tc_stage(logits, labels):
    n, c = logits.shape
    grid = (n + _BR - 1) // _BR
    n_pad = grid * _BR
    return pl.pallas_call(
        functools.partial(_conf_body, n),
        grid=(grid,),
        in_specs=[
            pl.BlockSpec((c, _BR), lambda i: (0, i)),
            pl.BlockSpec((_BR,), lambda i: (i,)),
        ],
        out_specs=pl.BlockSpec((_BR,), lambda i: (i,)),
        out_shape=jax.ShapeDtypeStruct((n_pad,), jnp.float32),
    )(logits.T, labels)


_NW = 16
_L = 16


def _sc_stage(sconf, n_real):
    n = sconf.shape[0]
    ch = n // _NW
    nvec = ch // _L
    assert ch * _NW == n and nvec * _L == ch and ch % 8 == 0

    uppers = [float(np.float32(i) / np.float32(_N_BINS))
              for i in range(1, _N_BINS)] + [1.0]
    inv_n = 1.0 / n_real

    mesh = plsc.VectorSubcoreMesh(core_axis_name="c", subcore_axis_name="s",
                                  num_cores=1)

    @functools.partial(
        pl.kernel,
        mesh=mesh,
        out_type=jax.ShapeDtypeStruct((_L,), jnp.float32),
        compiler_params=pltpu.CompilerParams(needs_layout_passes=False),
        scratch_types=[
            pltpu.VMEM((ch,), jnp.float32),
            pltpu.VMEM((3 * 16 * _L,), jnp.float32),
            pltpu.VMEM((_NW, 3 * 16 * _L), jnp.float32),
            pltpu.VMEM((_L,), jnp.float32),
            pltpu.VMEM_SHARED((_NW, 3 * 16 * _L), jnp.float32),
        ],
    )
    def sc_kernel(sconf_hbm, out_hbm, chunk_v, part_v, comb_v, outv_v, shared):
        wid = lax.axis_index("s")
        lane = lax.iota(jnp.int32, _L)
        zeros = jnp.zeros((_L,), jnp.float32)
        ones = jnp.ones((_L,), jnp.float32)
        one_i = jnp.ones((_L,), jnp.int32)
        zero_i = jnp.zeros((_L,), jnp.int32)

        def zk(k, _):
            part_v[pl.ds(k * _L, _L)] = zeros
            return 0
        lax.fori_loop(0, 3 * 16, zk, 0)

        pltpu.sync_copy(sconf_hbm.at[pl.ds(wid * ch, ch)], chunk_v)

        def body(t, _):
            v = chunk_v[pl.ds(t * _L, _L)]
            conf = jnp.abs(v)
            accf = jnp.where(v > 0, 1.0, 0.0)
            idx = zero_i
            for u in uppers:
                idx = idx + jnp.where(conf > u, one_i, zero_i)
            off = idx * _L + lane
            plsc.addupdate_scatter(part_v, [off], ones)
            plsc.addupdate_scatter(part_v, [off + 256], conf)
            plsc.addupdate_scatter(part_v, [off + 512], accf)
            return 0
        lax.fori_loop(0, nvec, body, 0)

        pltpu.sync_copy(part_v, shared.at[wid])
        plsc.subcore_barrier()

        @pl.when(wid == 0)
        def _():
            pltpu.sync_copy(shared, comb_v)

            def comb_k(k, _):
                def add_w(w, t):
                    return t + comb_v[w, pl.ds(k * _L, _L)]
                tot = lax.fori_loop(1, _NW, add_w, comb_v[0, pl.ds(k * _L, _L)])
                part_v[pl.ds(k * _L, _L)] = tot
                return 0
            lax.fori_loop(0, 3 * 16, comb_k, 0)

            vecs = []
            for a in range(3):
                vec = zeros
                for b in range(16):
                    s = jnp.sum(part_v[pl.ds(a * 256 + b * _L, _L)])
                    vec = jnp.where(lane == b, s, vec)
                vecs.append(vec)
            cntv, confv, accv = vecs
            denom = jnp.maximum(cntv, 1.0)
            avg_c = confv / denom
            avg_a = accv / denom
            prop = cntv * inv_n
            contrib = jnp.where(cntv > 0.0,
                                jnp.abs(avg_c - avg_a) * prop, 0.0)
            contrib = jnp.where(lane < _N_BINS, contrib, 0.0)
            outv_v[...] = jnp.sum(contrib) + zeros
            pltpu.sync_copy(outv_v, out_hbm)

    return sc_kernel(sconf)


def kernel(logits, labels):
    sconf = _tc_stage(logits, labels)
    out = _sc_stage(sconf, logits.shape[0])
    return out[:1]
